# baseline (speedup 1.0000x reference)
import jax
import jax.numpy as jnp
from jax.experimental import pallas as pl
from jax.experimental.pallas import tpu as pltpu

N = 4096
IN_F = 128
OUT_F = 128
HEADS = 2
HEAD_DIM = OUT_F // HEADS
ALPHA = 0.2
BLOCK_R = 256


def _gat_block_kernel(h_ref, adj_ref, w_ref, ap_ref, o_ref, wh_s, s_s, st_s,
                      whx_s):
    i = pl.program_id(0)

    @pl.when(i == 0)
    def _init():
        wh = jnp.dot(h_ref[:], w_ref[:], preferred_element_type=jnp.float32)
        wh_s[:] = wh
        s_s[:] = jnp.dot(wh, ap_ref[:], preferred_element_type=jnp.float32)
        st = jax.lax.dot_general(
            ap_ref[:], wh, (((0,), (1,)), ((), ())),
            preferred_element_type=jnp.float32)
        st_s[:] = st
        for hd in range(HEADS):
            dst = st[2 + hd:3 + hd, :]
            dmax = jnp.max(dst, axis=-1, keepdims=True)
            st_s[4 + hd:5 + hd, :] = jnp.exp(dst - dmax)
            st_s[6 + hd:7 + hd, :] = jnp.exp(ALPHA * (dst - dmax))
        col = jax.lax.broadcasted_iota(jnp.int32, (N, 64), 1)
        pad = jnp.where(col == 0, 1.0, 0.0).astype(jnp.bfloat16)
        for hd in range(HEADS):
            whx_s[:, hd * 128:hd * 128 + 64] = (
                wh[:, hd * HEAD_DIM:(hd + 1) * HEAD_DIM].astype(jnp.bfloat16))
            whx_s[:, hd * 128 + 64:(hd + 1) * 128] = pad

    adj = adj_ref[:].astype(jnp.bfloat16)
    for hd in range(HEADS):
        src = s_s[pl.ds(i * BLOCK_R, BLOCK_R), hd:hd + 1]
        dst = st_s[2 + hd:3 + hd, :]
        dmax = jnp.max(dst, axis=-1, keepdims=True)
        sd = src + dmax
        mt = jnp.maximum(sd, 0.0)
        a1 = jnp.exp(sd - mt).astype(jnp.bfloat16)
        a0 = jnp.exp(ALPHA * sd - mt).astype(jnp.bfloat16)
        b1 = st_s[4 + hd:5 + hd, :].astype(jnp.bfloat16)
        b0 = st_s[6 + hd:7 + hd, :].astype(jnp.bfloat16)
        p = adj * jnp.maximum(a1 * b1, a0 * b0)
        pwx = jnp.dot(p, whx_s[:, hd * 128:(hd + 1) * 128],
                      preferred_element_type=jnp.float32)
        pwh = pwx[:, 0:HEAD_DIM]
        s = pwx[:, HEAD_DIM:HEAD_DIM + 1]
        wh = wh_s[:, hd * HEAD_DIM:(hd + 1) * HEAD_DIM]
        mean_wh = jnp.mean(wh, axis=0, keepdims=True)
        out_h = jnp.where(s > 0, pwh / jnp.where(s > 0, s, 1.0), mean_wh)
        o_ref[:, hd * HEAD_DIM:(hd + 1) * HEAD_DIM] = jnp.where(
            out_h > 0, out_h, jnp.exp(out_h) - 1.0)


@jax.jit
def _run(h, adj, w2, apack):
    return pl.pallas_call(
        _gat_block_kernel,
        grid=(N // BLOCK_R,),
        in_specs=[
            pl.BlockSpec((N, IN_F), lambda i: (0, 0)),
            pl.BlockSpec((BLOCK_R, N), lambda i: (i, 0)),
            pl.BlockSpec((IN_F, OUT_F), lambda i: (0, 0)),
            pl.BlockSpec((IN_F, 128), lambda i: (0, 0)),
        ],
        out_specs=pl.BlockSpec((BLOCK_R, OUT_F), lambda i: (i, 0)),
        out_shape=jax.ShapeDtypeStruct((N, OUT_F), jnp.float32),
        scratch_shapes=[
            pltpu.VMEM((N, OUT_F), jnp.float32),
            pltpu.VMEM((N, 128), jnp.float32),
            pltpu.VMEM((128, N), jnp.float32),
            pltpu.VMEM((N, HEADS * 128), jnp.bfloat16),
        ],
    )(h, adj, w2, apack)


def kernel(h, adj, W, a_src, a_dst):
    w2 = jnp.concatenate([W[0], W[1]], axis=1)
    apack = jnp.zeros((IN_F, 128), jnp.float32)
    apack = apack.at[0:HEAD_DIM, 0].set(a_src[0, :, 0])
    apack = apack.at[HEAD_DIM:2 * HEAD_DIM, 1].set(a_src[1, :, 0])
    apack = apack.at[0:HEAD_DIM, 2].set(a_dst[0, :, 0])
    apack = apack.at[HEAD_DIM:2 * HEAD_DIM, 3].set(a_dst[1, :, 0])
    return _run(h, adj, w2, apack)


# speedup vs baseline: 1.2552x; 1.2552x over previous
import jax
import jax.numpy as jnp
from jax.experimental import pallas as pl
from jax.experimental.pallas import tpu as pltpu

N = 4096
IN_F = 128
OUT_F = 128
HEADS = 2
HEAD_DIM = OUT_F // HEADS
ALPHA = 0.2
BLOCK_R = 1024


def _gat_block_kernel(h_ref, adj_ref, w_ref, ap_ref, o_ref, wh_s, s_s, st_s,
                      whx_s):
    i = pl.program_id(0)

    @pl.when(i == 0)
    def _init():
        wh = jnp.dot(h_ref[:], w_ref[:], preferred_element_type=jnp.float32)
        wh_s[:] = wh
        s_s[:] = jnp.dot(wh, ap_ref[:], preferred_element_type=jnp.float32)
        st = jax.lax.dot_general(
            ap_ref[:], wh, (((0,), (1,)), ((), ())),
            preferred_element_type=jnp.float32)
        st_s[:] = st
        for hd in range(HEADS):
            dst = st[2 + hd:3 + hd, :]
            dmax = jnp.max(dst, axis=-1, keepdims=True)
            st_s[4 + hd:5 + hd, :] = jnp.exp(dst - dmax)
            st_s[6 + hd:7 + hd, :] = jnp.exp(ALPHA * (dst - dmax))
        col = jax.lax.broadcasted_iota(jnp.int32, (N, 64), 1)
        pad = jnp.where(col == 0, 1.0, 0.0).astype(jnp.bfloat16)
        for hd in range(HEADS):
            whx_s[:, hd * 128:hd * 128 + 64] = (
                wh[:, hd * HEAD_DIM:(hd + 1) * HEAD_DIM].astype(jnp.bfloat16))
            whx_s[:, hd * 128 + 64:(hd + 1) * 128] = pad

    adj = adj_ref[:].astype(jnp.bfloat16)
    for hd in range(HEADS):
        src = s_s[pl.ds(i * BLOCK_R, BLOCK_R), hd:hd + 1]
        dst = st_s[2 + hd:3 + hd, :]
        dmax = jnp.max(dst, axis=-1, keepdims=True)
        sd = src + dmax
        mt = jnp.maximum(sd, 0.0)
        a1 = jnp.exp(sd - mt).astype(jnp.bfloat16)
        a0 = jnp.exp(ALPHA * sd - mt).astype(jnp.bfloat16)
        b1 = st_s[4 + hd:5 + hd, :].astype(jnp.bfloat16)
        b0 = st_s[6 + hd:7 + hd, :].astype(jnp.bfloat16)
        p = adj * jnp.maximum(a1 * b1, a0 * b0)
        pwx = jnp.dot(p, whx_s[:, hd * 128:(hd + 1) * 128],
                      preferred_element_type=jnp.float32)
        pwh = pwx[:, 0:HEAD_DIM]
        s = pwx[:, HEAD_DIM:HEAD_DIM + 1]
        wh = wh_s[:, hd * HEAD_DIM:(hd + 1) * HEAD_DIM]
        mean_wh = jnp.mean(wh, axis=0, keepdims=True)
        out_h = jnp.where(s > 0, pwh / jnp.where(s > 0, s, 1.0), mean_wh)
        o_ref[:, hd * HEAD_DIM:(hd + 1) * HEAD_DIM] = jnp.where(
            out_h > 0, out_h, jnp.exp(out_h) - 1.0)


@jax.jit
def _run(h, adj, w2, apack):
    return pl.pallas_call(
        _gat_block_kernel,
        grid=(N // BLOCK_R,),
        in_specs=[
            pl.BlockSpec((N, IN_F), lambda i: (0, 0)),
            pl.BlockSpec((BLOCK_R, N), lambda i: (i, 0)),
            pl.BlockSpec((IN_F, OUT_F), lambda i: (0, 0)),
            pl.BlockSpec((IN_F, 128), lambda i: (0, 0)),
        ],
        out_specs=pl.BlockSpec((BLOCK_R, OUT_F), lambda i: (i, 0)),
        out_shape=jax.ShapeDtypeStruct((N, OUT_F), jnp.float32),
        scratch_shapes=[
            pltpu.VMEM((N, OUT_F), jnp.float32),
            pltpu.VMEM((N, 128), jnp.float32),
            pltpu.VMEM((128, N), jnp.float32),
            pltpu.VMEM((N, HEADS * 128), jnp.bfloat16),
        ],
    )(h, adj, w2, apack)


def kernel(h, adj, W, a_src, a_dst):
    w2 = jnp.concatenate([W[0], W[1]], axis=1)
    apack = jnp.zeros((IN_F, 128), jnp.float32)
    apack = apack.at[0:HEAD_DIM, 0].set(a_src[0, :, 0])
    apack = apack.at[HEAD_DIM:2 * HEAD_DIM, 1].set(a_src[1, :, 0])
    apack = apack.at[0:HEAD_DIM, 2].set(a_dst[0, :, 0])
    apack = apack.at[HEAD_DIM:2 * HEAD_DIM, 3].set(a_dst[1, :, 0])
    return _run(h, adj, w2, apack)


# speedup vs baseline: 1.3524x; 1.0774x over previous
import jax
import jax.numpy as jnp
from jax.experimental import pallas as pl
from jax.experimental.pallas import tpu as pltpu

N = 4096
IN_F = 128
OUT_F = 128
HEADS = 2
HEAD_DIM = OUT_F // HEADS
ALPHA = 0.2
BLOCK_R = 1024


def _gat_block_kernel(h_ref, adj_ref, w_ref, ap_ref, o_ref, s_s, st_s,
                      whx_s):
    i = pl.program_id(0)

    @pl.when(i == 0)
    def _init():
        wh = jnp.dot(h_ref[:], w_ref[:], preferred_element_type=jnp.float32)
        s_s[:] = jnp.dot(wh, ap_ref[:], preferred_element_type=jnp.float32)
        st = jax.lax.dot_general(
            ap_ref[:], wh, (((0,), (1,)), ((), ())),
            preferred_element_type=jnp.float32)
        st_s[:] = st
        for hd in range(HEADS):
            dst = st[2 + hd:3 + hd, :]
            dmax = jnp.max(dst, axis=-1, keepdims=True)
            st_s[4 + hd:5 + hd, :] = jnp.exp(dst - dmax)
            st_s[6 + hd:7 + hd, :] = jnp.exp(ALPHA * (dst - dmax))
            st_s[8 + hd:9 + hd, 0:N] = jnp.pad(
                jnp.mean(wh[:, hd * HEAD_DIM:(hd + 1) * HEAD_DIM], axis=0,
                         keepdims=True), ((0, 0), (0, N - HEAD_DIM)))
        col = jax.lax.broadcasted_iota(jnp.int32, (N, 64), 1)
        pad = jnp.where(col == 0, 1.0, 0.0).astype(jnp.bfloat16)
        for hd in range(HEADS):
            whx_s[:, hd * 128:hd * 128 + 64] = (
                wh[:, hd * HEAD_DIM:(hd + 1) * HEAD_DIM].astype(jnp.bfloat16))
            whx_s[:, hd * 128 + 64:(hd + 1) * 128] = pad

    adj = adj_ref[:].astype(jnp.bfloat16)
    for hd in range(HEADS):
        src = s_s[pl.ds(i * BLOCK_R, BLOCK_R), hd:hd + 1]
        dst = st_s[2 + hd:3 + hd, :]
        dmax = jnp.max(dst, axis=-1, keepdims=True)
        sd = src + dmax
        mt = jnp.maximum(sd, 0.0)
        a1 = jnp.exp(sd - mt).astype(jnp.bfloat16)
        a0 = jnp.exp(ALPHA * sd - mt).astype(jnp.bfloat16)
        b1 = st_s[4 + hd:5 + hd, :].astype(jnp.bfloat16)
        b0 = st_s[6 + hd:7 + hd, :].astype(jnp.bfloat16)
        p = adj * jnp.maximum(a1 * b1, a0 * b0)
        pwx = jnp.dot(p, whx_s[:, hd * 128:(hd + 1) * 128],
                      preferred_element_type=jnp.float32)
        pwh = pwx[:, 0:HEAD_DIM]
        s = pwx[:, HEAD_DIM:HEAD_DIM + 1]
        mean_wh = st_s[8 + hd:9 + hd, 0:HEAD_DIM]
        out_h = jnp.where(s > 0, pwh / jnp.where(s > 0, s, 1.0), mean_wh)
        o_ref[:, hd * HEAD_DIM:(hd + 1) * HEAD_DIM] = jnp.where(
            out_h > 0, out_h, jnp.exp(out_h) - 1.0)


@jax.jit
def _run(h, adj, w2, apack):
    return pl.pallas_call(
        _gat_block_kernel,
        grid=(N // BLOCK_R,),
        in_specs=[
            pl.BlockSpec((N, IN_F), lambda i: (0, 0)),
            pl.BlockSpec((BLOCK_R, N), lambda i: (i, 0)),
            pl.BlockSpec((IN_F, OUT_F), lambda i: (0, 0)),
            pl.BlockSpec((IN_F, 128), lambda i: (0, 0)),
        ],
        out_specs=pl.BlockSpec((BLOCK_R, OUT_F), lambda i: (i, 0)),
        out_shape=jax.ShapeDtypeStruct((N, OUT_F), jnp.float32),
        scratch_shapes=[
            pltpu.VMEM((N, 128), jnp.float32),
            pltpu.VMEM((128, N), jnp.float32),
            pltpu.VMEM((N, HEADS * 128), jnp.bfloat16),
        ],
    )(h, adj, w2, apack)


def kernel(h, adj, W, a_src, a_dst):
    w2 = jnp.concatenate([W[0], W[1]], axis=1)
    apack = jnp.zeros((IN_F, 128), jnp.float32)
    apack = apack.at[0:HEAD_DIM, 0].set(a_src[0, :, 0])
    apack = apack.at[HEAD_DIM:2 * HEAD_DIM, 1].set(a_src[1, :, 0])
    apack = apack.at[0:HEAD_DIM, 2].set(a_dst[0, :, 0])
    apack = apack.at[HEAD_DIM:2 * HEAD_DIM, 3].set(a_dst[1, :, 0])
    return _run(h, adj, w2, apack)
